# baseline (speedup 1.0000x reference)
import functools

import jax
import jax.numpy as jnp
from jax import lax
from jax.experimental import pallas as pl
from jax.experimental.pallas import tpu as pltpu
from jax.experimental.pallas import tpu_sc as plsc

N_NODES = 10000
NP = 10240
E = 320000
CH = 128
NCH = E // CH
EPW = E // 32
G = 64
TILES = 16
ROWS_PT = NP // TILES
R = 1024
NBLK = NP // R

_MESH = plsc.VectorSubcoreMesh(core_axis_name="c", subcore_axis_name="s")
_SC_PARAMS = pltpu.CompilerParams(needs_layout_passes=False)


def _settle(scratch_i32):
    plsc.subcore_barrier()

    def spin(i, a):
        return a * 3 + 1

    a = lax.fori_loop(0, 512, spin, jnp.zeros((16,), jnp.int32))
    scratch_i32[pl.ds(0, 16)] = a
    plsc.subcore_barrier()


def _deg_body(dst_hbm, out_hbm, hist, idxb, sem):
    c = lax.axis_index("c")
    s = lax.axis_index("s")
    w = c * TILES + s
    zero16 = jnp.zeros((16,), jnp.float32)

    def zero_it(i, carry):
        hist[pl.ds(i * 16, 16)] = zero16
        return carry

    lax.fori_loop(0, NP // 16, zero_it, 0)
    pltpu.sync_copy(dst_hbm.at[pl.ds(w * EPW, EPW)], idxb)
    one16 = jnp.ones((16,), jnp.float32)

    def it(i, carry):
        v = idxb[pl.ds(i * 16, 16)]
        plsc.addupdate_scatter(hist, [v], one16)
        return carry

    lax.fori_loop(0, EPW // 16, it, 0)
    pltpu.sync_copy(hist, out_hbm.at[w])


_deg_kernel = pl.kernel(
    _deg_body,
    out_type=jax.ShapeDtypeStruct((32, NP), jnp.float32),
    mesh=_MESH,
    compiler_params=_SC_PARAMS,
    scratch_types=[
        pltpu.VMEM((NP,), jnp.float32),
        pltpu.VMEM((EPW,), jnp.int32),
        pltpu.SemaphoreType.DMA,
    ],
)


def _agg1_body(g_hbm, zq_hbm, src_hbm, dst_hbm, out_hbm, accum, idx_s, idx_d,
               rows_v, sem):
    c = lax.axis_index("c")
    s = lax.axis_index("s")
    base = s * ROWS_PT

    @pl.when(c == 0)
    def _():
        pltpu.sync_copy(g_hbm.at[pl.ds(base, ROWS_PT)],
                        accum.at[pl.ds(base, ROWS_PT)])

    @pl.when(c == 1)
    def _():
        pltpu.sync_copy(zq_hbm.at[pl.ds(base, ROWS_PT)],
                        accum.at[pl.ds(base, ROWS_PT)])

    plsc.subcore_barrier()
    half = NCH // 2
    ntr = 78 + jnp.where(s < half - 16 * 78, 1, 0)

    def it(i, carry):
        r = c * half + s + i * TILES
        pltpu.sync_copy(src_hbm.at[r], idx_s)
        pltpu.sync_copy(dst_hbm.at[r], idx_d)
        pltpu.async_copy(g_hbm.at[idx_s], rows_v, sem).wait()
        pltpu.sync_copy(rows_v, accum.at[idx_d], add=True)
        return carry

    lax.fori_loop(0, ntr, it, 0)
    _settle(idx_s)
    pltpu.sync_copy(accum.at[pl.ds(base, ROWS_PT)],
                    out_hbm.at[pl.ds(c * NP + base, ROWS_PT)])


_agg1_kernel = pl.kernel(
    _agg1_body,
    out_type=jax.ShapeDtypeStruct((2 * NP, 128), jnp.float32),
    mesh=_MESH,
    compiler_params=_SC_PARAMS,
    scratch_types=[
        pltpu.VMEM_SHARED((NP, 128), jnp.float32),
        pltpu.VMEM((CH,), jnp.int32),
        pltpu.VMEM((CH,), jnp.int32),
        pltpu.VMEM((CH, 128), jnp.float32),
        pltpu.SemaphoreType.DMA,
    ],
)


def _agg2_body(g_hbm, src_hbm, dst_hbm, out_hbm, accum, idx_s, idx_d,
               rows_v, sem):
    c = lax.axis_index("c")
    s = lax.axis_index("s")
    base = s * ROWS_PT
    pltpu.sync_copy(g_hbm.at[pl.ds(c * NP + base, ROWS_PT)],
                    accum.at[pl.ds(base, ROWS_PT)])
    plsc.subcore_barrier()
    ntr = 156 + jnp.where(s < NCH - 16 * 156, 1, 0)

    def it(i, carry):
        r = s + i * TILES
        pltpu.sync_copy(src_hbm.at[c * NCH + r], idx_s)
        pltpu.sync_copy(dst_hbm.at[r], idx_d)
        pltpu.async_copy(g_hbm.at[idx_s], rows_v, sem).wait()
        pltpu.sync_copy(rows_v, accum.at[idx_d], add=True)
        return carry

    lax.fori_loop(0, ntr, it, 0)
    _settle(idx_s)
    pltpu.sync_copy(accum.at[pl.ds(base, ROWS_PT)],
                    out_hbm.at[pl.ds(c * NP + base, ROWS_PT)])


_agg2_kernel = pl.kernel(
    _agg2_body,
    out_type=jax.ShapeDtypeStruct((2 * NP, 128), jnp.float32),
    mesh=_MESH,
    compiler_params=_SC_PARAMS,
    scratch_types=[
        pltpu.VMEM_SHARED((NP, 128), jnp.float32),
        pltpu.VMEM((CH,), jnp.int32),
        pltpu.VMEM((CH,), jnp.int32),
        pltpu.VMEM((CH, 128), jnp.float32),
        pltpu.SemaphoreType.DMA,
    ],
)


def _deg_dinv(d_ref):
    deg = jnp.sum(d_ref[...], axis=0) + 1.0
    return lax.rsqrt(deg)


def _stage1_body(x_ref, w_ref, d_ref, out_ref):
    h = jnp.dot(x_ref[...], w_ref[...], preferred_element_type=jnp.float32)
    out_ref[...] = h * _deg_dinv(d_ref)[:, None]


def _stage1(x_pad, W1, dparts):
    return pl.pallas_call(
        _stage1_body,
        grid=(NBLK,),
        in_specs=[
            pl.BlockSpec((R, 128), lambda i: (i, 0)),
            pl.BlockSpec((128, 128), lambda i: (0, 0)),
            pl.BlockSpec((32, R), lambda i: (0, i)),
        ],
        out_specs=pl.BlockSpec((R, 128), lambda i: (i, 0)),
        out_shape=jax.ShapeDtypeStruct((NP, 128), jnp.float32),
    )(x_pad, W1, dparts)


def _pool_body(mode, d_out, has_next, *refs):
    if has_next:
        (s0_ref, s1_ref, d_ref, b_ref, bt_ref, w_ref,
         p_ref, cnt_ref, g_ref) = refs
    else:
        s0_ref, s1_ref, d_ref, b_ref, bt_ref, p_ref, cnt_ref = refs
    i = pl.program_id(0)
    if mode == "sum":
        S = s0_ref[...] + s1_ref[...]
    else:
        S = jnp.concatenate([s0_ref[...], s1_ref[...]], axis=1)
    dinv = _deg_dinv(d_ref)
    h = jnp.maximum(S * dinv[:, None] + b_ref[...], 0.0)
    bt = bt_ref[0, 0]
    oh = (lax.broadcasted_iota(jnp.int32, (G, R), 0) == bt[None, :]
          ).astype(jnp.float32)
    pp = jnp.dot(oh, h, preferred_element_type=jnp.float32)
    cc = jnp.broadcast_to(jnp.sum(oh, axis=1)[:, None], (G, d_out))

    @pl.when(i == 0)
    def _():
        p_ref[...] = pp
        cnt_ref[...] = cc

    @pl.when(i > 0)
    def _():
        p_ref[...] += pp
        cnt_ref[...] += cc

    if has_next:
        g2 = jnp.dot(h, w_ref[...], preferred_element_type=jnp.float32) \
            * dinv[:, None]
        g_ref[0] = g2[:, :128]
        g_ref[1] = g2[:, 128:]

    @pl.when(i == NBLK - 1)
    def _():
        p_ref[...] = p_ref[...] / jnp.maximum(cnt_ref[...], 1.0)


def _pool_l1(S0, S1, dparts, b1, bt3d, W2):
    return pl.pallas_call(
        functools.partial(_pool_body, "sum", 128, True),
        grid=(NBLK,),
        in_specs=[
            pl.BlockSpec((R, 128), lambda i: (i, 0)),
            pl.BlockSpec((R, 128), lambda i: (i, 0)),
            pl.BlockSpec((32, R), lambda i: (0, i)),
            pl.BlockSpec((1, 128), lambda i: (0, 0)),
            pl.BlockSpec((1, 1, R), lambda i: (i, 0, 0)),
            pl.BlockSpec((128, 256), lambda i: (0, 0)),
        ],
        out_specs=[
            pl.BlockSpec((G, 128), lambda i: (0, 0)),
            pl.BlockSpec((G, 128), lambda i: (0, 0)),
            pl.BlockSpec((2, R, 128), lambda i: (0, i, 0)),
        ],
        out_shape=[
            jax.ShapeDtypeStruct((G, 128), jnp.float32),
            jax.ShapeDtypeStruct((G, 128), jnp.float32),
            jax.ShapeDtypeStruct((2, NP, 128), jnp.float32),
        ],
    )(S0, S1, dparts, b1, bt3d, W2)


def _pool_l2(S2a, S2b, dparts, b2, bt3d):
    return pl.pallas_call(
        functools.partial(_pool_body, "concat", 256, False),
        grid=(NBLK,),
        in_specs=[
            pl.BlockSpec((R, 128), lambda i: (i, 0)),
            pl.BlockSpec((R, 128), lambda i: (i, 0)),
            pl.BlockSpec((32, R), lambda i: (0, i)),
            pl.BlockSpec((1, 256), lambda i: (0, 0)),
            pl.BlockSpec((1, 1, R), lambda i: (i, 0, 0)),
        ],
        out_specs=[
            pl.BlockSpec((G, 256), lambda i: (0, 0)),
            pl.BlockSpec((G, 256), lambda i: (0, 0)),
        ],
        out_shape=[
            jax.ShapeDtypeStruct((G, 256), jnp.float32),
            jax.ShapeDtypeStruct((G, 256), jnp.float32),
        ],
    )(S2a, S2b, dparts, b2, bt3d)


def kernel(x, edge_index, edge_weight, batch, W1, b1, W2, b2):
    src2d = edge_index[0].reshape(NCH, CH)
    dst2d = edge_index[1].reshape(NCH, CH)
    src_stack = jnp.concatenate([src2d, src2d + NP], axis=0)
    x_pad = jnp.pad(x, ((0, NP - N_NODES), (0, 0)))
    bt3d = jnp.pad(batch, (0, NP - N_NODES), constant_values=G
                   ).reshape(NBLK, 1, R)
    zeros_q = jnp.zeros((NP, 128), jnp.float32)

    dparts = _deg_kernel(edge_index[1])

    g1 = _stage1(x_pad, W1, dparts)
    S1 = _agg1_kernel(g1, zeros_q, src2d, dst2d)

    p1, _, g2 = _pool_l1(S1[:NP], S1[NP:], dparts,
                         b1.reshape(1, 128), bt3d, W2)
    g2 = g2.reshape(2 * NP, 128)
    S2 = _agg2_kernel(g2, src_stack, dst2d)

    p2, _ = _pool_l2(S2[:NP], S2[NP:], dparts,
                     b2.reshape(1, 256), bt3d)
    return (x, p1, p2)


# speedup vs baseline: 1.1362x; 1.1362x over previous
import functools

import jax
import jax.numpy as jnp
from jax import lax
from jax.experimental import pallas as pl
from jax.experimental.pallas import tpu as pltpu
from jax.experimental.pallas import tpu_sc as plsc

N_NODES = 10000
NP = 10240
E = 320000
CH = 128
NCH = E // CH
EPW = E // 32
G = 64
TILES = 16
ROWS_PT = NP // TILES
R = 1024
NBLK = NP // R

_MESH = plsc.VectorSubcoreMesh(core_axis_name="c", subcore_axis_name="s")
_SC_PARAMS = pltpu.CompilerParams(needs_layout_passes=False)


def _settle(scratch_i32):
    plsc.subcore_barrier()

    def spin(i, a):
        return a * 3 + 1

    a = lax.fori_loop(0, 512, spin, jnp.zeros((16,), jnp.int32))
    scratch_i32[pl.ds(0, 16)] = a
    plsc.subcore_barrier()


def _deg_body(dst_hbm, out_hbm, hist, idxb, sem):
    c = lax.axis_index("c")
    s = lax.axis_index("s")
    w = c * TILES + s
    zero16 = jnp.zeros((16,), jnp.float32)

    def zero_it(i, carry):
        hist[pl.ds(i * 16, 16)] = zero16
        return carry

    lax.fori_loop(0, NP // 16, zero_it, 0)
    pltpu.sync_copy(dst_hbm.at[pl.ds(w * EPW, EPW)], idxb)
    one16 = jnp.ones((16,), jnp.float32)

    def it(i, carry):
        v = idxb[pl.ds(i * 16, 16)]
        plsc.addupdate_scatter(hist, [v], one16)
        return carry

    lax.fori_loop(0, EPW // 16, it, 0)
    pltpu.sync_copy(hist, out_hbm.at[w])


_deg_kernel = pl.kernel(
    _deg_body,
    out_type=jax.ShapeDtypeStruct((32, NP), jnp.float32),
    mesh=_MESH,
    compiler_params=_SC_PARAMS,
    scratch_types=[
        pltpu.VMEM((NP,), jnp.float32),
        pltpu.VMEM((EPW,), jnp.int32),
        pltpu.SemaphoreType.DMA,
    ],
)


def _agg1_body(g_hbm, zq_hbm, es_hbm, out_hbm, accum, ip, rows_v, sem):
    c = lax.axis_index("c")
    s = lax.axis_index("s")
    base = s * ROWS_PT

    @pl.when(c == 0)
    def _():
        pltpu.sync_copy(g_hbm.at[pl.ds(base, ROWS_PT)],
                        accum.at[pl.ds(base, ROWS_PT)])

    @pl.when(c == 1)
    def _():
        pltpu.sync_copy(zq_hbm.at[pl.ds(base, ROWS_PT)],
                        accum.at[pl.ds(base, ROWS_PT)])

    plsc.subcore_barrier()
    half = NCH // 2
    ntr = 78 + jnp.where(s < half - 16 * 78, 1, 0)

    def it(i, carry):
        r = c * half + s + i * TILES
        pltpu.sync_copy(es_hbm.at[pl.ds(2 * r, 2)], ip)
        pltpu.async_copy(g_hbm.at[ip.at[0]], rows_v, sem).wait()
        pltpu.sync_copy(rows_v, accum.at[ip.at[1]], add=True)
        return carry

    lax.fori_loop(0, ntr, it, 0)
    _settle(ip.at[0])
    pltpu.sync_copy(accum.at[pl.ds(base, ROWS_PT)],
                    out_hbm.at[pl.ds(c * NP + base, ROWS_PT)])


_agg1_kernel = pl.kernel(
    _agg1_body,
    out_type=jax.ShapeDtypeStruct((2 * NP, 128), jnp.float32),
    mesh=_MESH,
    compiler_params=_SC_PARAMS,
    scratch_types=[
        pltpu.VMEM_SHARED((NP, 128), jnp.float32),
        pltpu.VMEM((2, CH), jnp.int32),
        pltpu.VMEM((CH, 128), jnp.float32),
        pltpu.SemaphoreType.DMA,
    ],
)


def _agg2_body(g_hbm, es_hbm, out_hbm, accum, ip, rows_v, sem):
    c = lax.axis_index("c")
    s = lax.axis_index("s")
    base = s * ROWS_PT
    pltpu.sync_copy(g_hbm.at[pl.ds(c * NP + base, ROWS_PT)],
                    accum.at[pl.ds(base, ROWS_PT)])
    plsc.subcore_barrier()
    ntr = 156 + jnp.where(s < NCH - 16 * 156, 1, 0)

    def it(i, carry):
        r = s + i * TILES
        pltpu.sync_copy(es_hbm.at[pl.ds(2 * (c * NCH + r), 2)], ip)
        pltpu.async_copy(g_hbm.at[ip.at[0]], rows_v, sem).wait()
        pltpu.sync_copy(rows_v, accum.at[ip.at[1]], add=True)
        return carry

    lax.fori_loop(0, ntr, it, 0)
    _settle(ip.at[0])
    pltpu.sync_copy(accum.at[pl.ds(base, ROWS_PT)],
                    out_hbm.at[pl.ds(c * NP + base, ROWS_PT)])


_agg2_kernel = pl.kernel(
    _agg2_body,
    out_type=jax.ShapeDtypeStruct((2 * NP, 128), jnp.float32),
    mesh=_MESH,
    compiler_params=_SC_PARAMS,
    scratch_types=[
        pltpu.VMEM_SHARED((NP, 128), jnp.float32),
        pltpu.VMEM((2, CH), jnp.int32),
        pltpu.VMEM((CH, 128), jnp.float32),
        pltpu.SemaphoreType.DMA,
    ],
)


def _deg_dinv(d_ref):
    deg = jnp.sum(d_ref[...], axis=0) + 1.0
    return lax.rsqrt(deg)


def _stage1_body(x_ref, w_ref, d_ref, out_ref):
    h = jnp.dot(x_ref[...], w_ref[...], preferred_element_type=jnp.float32)
    out_ref[...] = h * _deg_dinv(d_ref)[:, None]


def _stage1(x_pad, W1, dparts):
    return pl.pallas_call(
        _stage1_body,
        grid=(NBLK,),
        in_specs=[
            pl.BlockSpec((R, 128), lambda i: (i, 0)),
            pl.BlockSpec((128, 128), lambda i: (0, 0)),
            pl.BlockSpec((32, R), lambda i: (0, i)),
        ],
        out_specs=pl.BlockSpec((R, 128), lambda i: (i, 0)),
        out_shape=jax.ShapeDtypeStruct((NP, 128), jnp.float32),
    )(x_pad, W1, dparts)


def _pool_body(mode, d_out, has_next, *refs):
    if has_next:
        (s0_ref, s1_ref, d_ref, b_ref, bt_ref, w_ref,
         p_ref, cnt_ref, g_ref) = refs
    else:
        s0_ref, s1_ref, d_ref, b_ref, bt_ref, p_ref, cnt_ref = refs
    i = pl.program_id(0)
    if mode == "sum":
        S = s0_ref[...] + s1_ref[...]
    else:
        S = jnp.concatenate([s0_ref[...], s1_ref[...]], axis=1)
    dinv = _deg_dinv(d_ref)
    h = jnp.maximum(S * dinv[:, None] + b_ref[...], 0.0)
    bt = bt_ref[0, 0]
    oh = (lax.broadcasted_iota(jnp.int32, (G, R), 0) == bt[None, :]
          ).astype(jnp.float32)
    pp = jnp.dot(oh, h, preferred_element_type=jnp.float32)
    cc = jnp.broadcast_to(jnp.sum(oh, axis=1)[:, None], (G, d_out))

    @pl.when(i == 0)
    def _():
        p_ref[...] = pp
        cnt_ref[...] = cc

    @pl.when(i > 0)
    def _():
        p_ref[...] += pp
        cnt_ref[...] += cc

    if has_next:
        g2 = jnp.dot(h, w_ref[...], preferred_element_type=jnp.float32) \
            * dinv[:, None]
        g_ref[0] = g2[:, :128]
        g_ref[1] = g2[:, 128:]

    @pl.when(i == NBLK - 1)
    def _():
        p_ref[...] = p_ref[...] / jnp.maximum(cnt_ref[...], 1.0)


def _pool_l1(S0, S1, dparts, b1, bt3d, W2):
    return pl.pallas_call(
        functools.partial(_pool_body, "sum", 128, True),
        grid=(NBLK,),
        in_specs=[
            pl.BlockSpec((R, 128), lambda i: (i, 0)),
            pl.BlockSpec((R, 128), lambda i: (i, 0)),
            pl.BlockSpec((32, R), lambda i: (0, i)),
            pl.BlockSpec((1, 128), lambda i: (0, 0)),
            pl.BlockSpec((1, 1, R), lambda i: (i, 0, 0)),
            pl.BlockSpec((128, 256), lambda i: (0, 0)),
        ],
        out_specs=[
            pl.BlockSpec((G, 128), lambda i: (0, 0)),
            pl.BlockSpec((G, 128), lambda i: (0, 0)),
            pl.BlockSpec((2, R, 128), lambda i: (0, i, 0)),
        ],
        out_shape=[
            jax.ShapeDtypeStruct((G, 128), jnp.float32),
            jax.ShapeDtypeStruct((G, 128), jnp.float32),
            jax.ShapeDtypeStruct((2, NP, 128), jnp.float32),
        ],
    )(S0, S1, dparts, b1, bt3d, W2)


def _pool_l2(S2a, S2b, dparts, b2, bt3d):
    return pl.pallas_call(
        functools.partial(_pool_body, "concat", 256, False),
        grid=(NBLK,),
        in_specs=[
            pl.BlockSpec((R, 128), lambda i: (i, 0)),
            pl.BlockSpec((R, 128), lambda i: (i, 0)),
            pl.BlockSpec((32, R), lambda i: (0, i)),
            pl.BlockSpec((1, 256), lambda i: (0, 0)),
            pl.BlockSpec((1, 1, R), lambda i: (i, 0, 0)),
        ],
        out_specs=[
            pl.BlockSpec((G, 256), lambda i: (0, 0)),
            pl.BlockSpec((G, 256), lambda i: (0, 0)),
        ],
        out_shape=[
            jax.ShapeDtypeStruct((G, 256), jnp.float32),
            jax.ShapeDtypeStruct((G, 256), jnp.float32),
        ],
    )(S2a, S2b, dparts, b2, bt3d)


def kernel(x, edge_index, edge_weight, batch, W1, b1, W2, b2):
    src2d = edge_index[0].reshape(NCH, CH)
    dst2d = edge_index[1].reshape(NCH, CH)
    esi1 = jnp.stack([src2d, dst2d], axis=1).reshape(2 * NCH, CH)
    esi2 = jnp.concatenate(
        [esi1, jnp.stack([src2d + NP, dst2d], axis=1).reshape(2 * NCH, CH)],
        axis=0)
    x_pad = jnp.pad(x, ((0, NP - N_NODES), (0, 0)))
    bt3d = jnp.pad(batch, (0, NP - N_NODES), constant_values=G
                   ).reshape(NBLK, 1, R)
    zeros_q = jnp.zeros((NP, 128), jnp.float32)

    dparts = _deg_kernel(edge_index[1])

    g1 = _stage1(x_pad, W1, dparts)
    S1 = _agg1_kernel(g1, zeros_q, esi1)

    p1, _, g2 = _pool_l1(S1[:NP], S1[NP:], dparts,
                         b1.reshape(1, 128), bt3d, W2)
    g2 = g2.reshape(2 * NP, 128)
    S2 = _agg2_kernel(g2, esi2)

    p2, _ = _pool_l2(S2[:NP], S2[NP:], dparts,
                     b2.reshape(1, 256), bt3d)
    return (x, p1, p2)
